# baseline (speedup 1.0000x reference)
import functools

import jax
import jax.numpy as jnp
from jax import lax
from jax.experimental import pallas as pl
from jax.experimental.pallas import tpu as pltpu
from jax.experimental.pallas import tpu_sc as plsc

B, N, DIN, H, L, K = 16, 100, 2, 128, 3, 10
NP = 104
NJ = 112
NI = 112
NCH = NJ // 16
NROWS = B * NI
NWORK = 32
RPW = NROWS // NWORK


def _splat_min(v):
    iota = lax.iota(jnp.int32, 16)
    for sh in (8, 4, 2, 1):
        idx = jnp.bitwise_xor(iota, sh)
        v = jnp.minimum(v, v.at[idx].get(mode="promise_in_bounds"))
    return v


def _sc_knn_body(dis_hbm, m_hbm, a_hbm, d_v, m_v, a_v):
    f32 = jnp.float32
    wid = lax.axis_index("s") * 2 + lax.axis_index("c")
    base = wid * RPW
    pltpu.sync_copy(dis_hbm.at[pl.ds(base, RPW)], d_v)

    iota = lax.iota(jnp.int32, 16)
    jids = [iota + (cc * 16) for cc in range(NCH)]

    UNROLL = 4

    def row_body(rr, carry):
        for u in range(UNROLL):
            r = rr * UNROLL + u
            vs = [d_v[r, pl.ds(cc * 16, 16)] for cc in range(NCH)]
            ms = [jnp.zeros((16,), f32) for _ in range(NCH)]
            for k in range(K + 1):
                mn = vs[0]
                for cc in range(1, NCH):
                    mn = jnp.minimum(mn, vs[cc])
                mval = _splat_min(mn)
                cands = [jnp.where(vs[cc] == mval, jids[cc], 9999)
                         for cc in range(NCH)]
                cm = cands[0]
                for cc in range(1, NCH):
                    cm = jnp.minimum(cm, cands[cc])
                jstar = _splat_min(cm)
                sels = [jids[cc] == jstar for cc in range(NCH)]
                vs = [jnp.where(sels[cc], 9.0, vs[cc]) for cc in range(NCH)]
                if k > 0:
                    ms = [jnp.where(sels[cc], 1.0, ms[cc])
                          for cc in range(NCH)]
            i_node = (base + r) % NI
            for cc in range(NCH):
                m_v[r, pl.ds(cc * 16, 16)] = ms[cc]
                a_v[r, pl.ds(cc * 16, 16)] = jnp.where(jids[cc] == i_node,
                                                       -1.0, ms[cc])
        return carry

    lax.fori_loop(0, RPW // UNROLL, row_body, 0)
    pltpu.sync_copy(m_v, m_hbm.at[pl.ds(base, RPW)])
    pltpu.sync_copy(a_v, a_hbm.at[pl.ds(base, RPW)])


def _sc_knn(dis_sc):
    f32 = jnp.float32
    mesh = plsc.VectorSubcoreMesh(core_axis_name="c", subcore_axis_name="s")
    fn = functools.partial(
        pl.kernel,
        mesh=mesh,
        out_type=[jax.ShapeDtypeStruct((NROWS, NJ), f32),
                  jax.ShapeDtypeStruct((NROWS, NJ), f32)],
        scratch_types=[pltpu.VMEM((RPW, NJ), f32),
                       pltpu.VMEM((RPW, NJ), f32),
                       pltpu.VMEM((RPW, NJ), f32)],
    )(_sc_knn_body)
    return fn(dis_sc)


IPP = 2


def _gcn_body(node_ref, dem_ref, m_ref, a_ref, disf_ref, W1_ref, b1_ref,
              W23_ref, w3_ref, b23_ref, w4_ref, w5_ref, b45_ref, Wn_ref,
              We_ref, Wl_self_ref, bl_self_ref, Wl_nb_ref, bl_nb_ref,
              Wl_e_ref, bl_e_ref, Wl_from_ref, Wl_to_ref,
              x_out_ref, e_out_ref):
    f32 = jnp.float32
    for s in range(IPP):
        node = node_ref[s]
        dem = dem_ref[s]
        M = m_ref[s][:NP, :NP]
        a = a_ref[s][:NP, :NP]

        xd = jax.nn.relu(jnp.dot(node, W1_ref[...],
                                 preferred_element_type=f32) + b1_ref[...])
        xc = jax.nn.relu(jnp.dot(node, W23_ref[...], preferred_element_type=f32)
                         + dem * w3_ref[...] + b23_ref[...])
        row0 = lax.broadcasted_iota(jnp.int32, (NP, 1), 0) == 0
        x = jnp.where(row0, xd, xc)
        x = jnp.dot(x, Wn_ref[...], preferred_element_type=f32)

        a_part = (a[:, :, None] * w5_ref[...][None]).reshape(NP * NP, H)
        e0 = jax.nn.relu(disf_ref[s] * w4_ref[...] + a_part + b45_ref[...])
        e = jnp.dot(e0, We_ref[...], preferred_element_type=f32)

        inv_k = 1.0 / K
        for l in range(L):
            mx = jnp.dot(M, x, preferred_element_type=f32) * inv_k
            agg = jnp.dot(mx, Wl_nb_ref[l], preferred_element_type=f32) + bl_nb_ref[l]
            x = x + jax.nn.relu(
                jnp.dot(x, Wl_self_ref[l], preferred_element_type=f32)
                + bl_self_ref[l] + agg)
            xf = jnp.dot(x, Wl_from_ref[l], preferred_element_type=f32) + bl_e_ref[l]
            xt = jnp.dot(x, Wl_to_ref[l], preferred_element_type=f32)
            et = jnp.dot(e, Wl_e_ref[l], preferred_element_type=f32)
            e3 = et.reshape(NP, NP, H) + xf[:, None, :] + xt[None, :, :]
            e = e + jax.nn.relu(e3).reshape(NP * NP, H)

        x_out_ref[s] = x[:N]
        e3 = e.reshape(NP, NP, H)
        for i in range(N):
            e_out_ref[pl.ds((s * N + i) * N, N), :] = e3[i, :N, :]


def kernel(node, demand, dis, W1, b1, W2, b2, W3, b3, W4, b4, W5, b5, Wn, We,
           Wl_self, bl_self, Wl_nb, bl_nb, Wl_e, bl_e, Wl_from, Wl_to):
    f32 = jnp.float32
    Hh = H // 2
    W23 = jnp.zeros((DIN, H), f32).at[:, :Hh].set(W2)
    w3 = jnp.zeros((1, H), f32).at[0, Hh:].set(W3[0])
    b23 = jnp.concatenate([b2, b3])[None]
    w4 = jnp.concatenate([W4[0], jnp.zeros((Hh,), f32)])[None]
    w5 = jnp.concatenate([jnp.zeros((Hh,), f32), W5[0]])[None]
    b45 = jnp.concatenate([b4, b5])[None]

    P = NP - N
    dis_p = jnp.pad(dis, ((0, 0), (0, P), (0, P)), constant_values=2.0)
    dis_flat = dis_p.reshape(B, NP * NP, 1)
    dis_sc = jnp.pad(dis_p, ((0, 0), (0, NI - NP), (0, NJ - NP)),
                     constant_values=2.0).reshape(NROWS, NJ)
    node_p = jnp.pad(node, ((0, 0), (0, P), (0, 0)))
    dem_p = jnp.pad(demand, ((0, 0), (0, P)))[..., None]

    m_all, a_all = _sc_knn(dis_sc)
    m3 = m_all.reshape(B, NI, NJ)
    a3 = a_all.reshape(B, NI, NJ)

    rep = lambda shape: pl.BlockSpec(shape, lambda b: (0,) * len(shape))

    grid_spec = pl.GridSpec(
        grid=(B // IPP,),
        in_specs=[
            pl.BlockSpec((IPP, NP, DIN), lambda b: (b, 0, 0)),
            pl.BlockSpec((IPP, NP, 1), lambda b: (b, 0, 0)),
            pl.BlockSpec((IPP, NI, NJ), lambda b: (b, 0, 0)),
            pl.BlockSpec((IPP, NI, NJ), lambda b: (b, 0, 0)),
            pl.BlockSpec((IPP, NP * NP, 1), lambda b: (b, 0, 0)),
            rep((DIN, H)), rep((1, H)), rep((DIN, H)), rep((1, H)),
            rep((1, H)), rep((1, H)), rep((1, H)), rep((1, H)),
            rep((H, H)), rep((H, H)),
            rep((L, H, H)), rep((L, H)), rep((L, H, H)), rep((L, H)),
            rep((L, H, H)), rep((L, H)), rep((L, H, H)), rep((L, H, H)),
        ],
        out_specs=[
            pl.BlockSpec((IPP, N, H), lambda b: (b, 0, 0)),
            pl.BlockSpec((IPP * N * N, H), lambda b: (b, 0)),
        ],
    )

    x_out, e_out = pl.pallas_call(
        _gcn_body,
        grid_spec=grid_spec,
        out_shape=[
            jax.ShapeDtypeStruct((B, N, H), f32),
            jax.ShapeDtypeStruct((B * N * N, H), f32),
        ],
        compiler_params=pltpu.CompilerParams(
            dimension_semantics=("arbitrary",),
        ),
    )(node_p, dem_p, m3, a3, dis_flat, W1, b1[None], W23, w3, b23, w4, w5, b45,
      Wn, We, Wl_self, bl_self, Wl_nb, bl_nb, Wl_e, bl_e, Wl_from, Wl_to)
    return (x_out, e_out.reshape(B, N, N, H))


# speedup vs baseline: 1.0133x; 1.0133x over previous
import functools

import jax
import jax.numpy as jnp
from jax import lax
from jax.experimental import pallas as pl
from jax.experimental.pallas import tpu as pltpu
from jax.experimental.pallas import tpu_sc as plsc

B, N, DIN, H, L, K = 16, 100, 2, 128, 3, 10
NP = 104
NJ = 112
NI = 112
NCH = NJ // 16
NROWS = B * NI
NWORK = 32
RPW = NROWS // NWORK


def _splat_min(v):
    iota = lax.iota(jnp.int32, 16)
    for sh in (8, 4, 2, 1):
        idx = jnp.bitwise_xor(iota, sh)
        v = jnp.minimum(v, v.at[idx].get(mode="promise_in_bounds"))
    return v


def _sc_knn_body(dis_hbm, m_hbm, d_v, m_v):
    f32 = jnp.float32
    wid = lax.axis_index("s") * 2 + lax.axis_index("c")
    base = wid * RPW
    pltpu.sync_copy(dis_hbm.at[pl.ds(base, RPW)], d_v)

    iota = lax.iota(jnp.int32, 16)
    jids = [iota + (cc * 16) for cc in range(NCH)]

    UNROLL = 4

    def row_body(rr, carry):
        for u in range(UNROLL):
            r = rr * UNROLL + u
            vs = [d_v[r, pl.ds(cc * 16, 16)] for cc in range(NCH)]
            ms = [jnp.zeros((16,), f32) for _ in range(NCH)]
            for k in range(K + 1):
                mn = vs[0]
                for cc in range(1, NCH):
                    mn = jnp.minimum(mn, vs[cc])
                mval = _splat_min(mn)
                cands = [jnp.where(vs[cc] == mval, jids[cc], 9999)
                         for cc in range(NCH)]
                cm = cands[0]
                for cc in range(1, NCH):
                    cm = jnp.minimum(cm, cands[cc])
                jstar = _splat_min(cm)
                sels = [jids[cc] == jstar for cc in range(NCH)]
                vs = [jnp.where(sels[cc], 9.0, vs[cc]) for cc in range(NCH)]
                if k > 0:
                    ms = [jnp.where(sels[cc], 1.0, ms[cc])
                          for cc in range(NCH)]
            for cc in range(NCH):
                m_v[r, pl.ds(cc * 16, 16)] = ms[cc]
        return carry

    lax.fori_loop(0, RPW // UNROLL, row_body, 0)
    pltpu.sync_copy(m_v, m_hbm.at[pl.ds(base, RPW)])


def _sc_knn(dis_sc):
    f32 = jnp.float32
    mesh = plsc.VectorSubcoreMesh(core_axis_name="c", subcore_axis_name="s")
    fn = functools.partial(
        pl.kernel,
        mesh=mesh,
        out_type=[jax.ShapeDtypeStruct((NROWS, NJ), f32)],
        scratch_types=[pltpu.VMEM((RPW, NJ), f32),
                       pltpu.VMEM((RPW, NJ), f32)],
    )(_sc_knn_body)
    return fn(dis_sc)


IPP = 2


def _gcn_body(node_ref, dem_ref, m_ref, disf_ref, W1_ref, b1_ref,
              W23_ref, w3_ref, b23_ref, w4_ref, w5_ref, b45_ref, Wn_ref,
              We_ref, Wl_self_ref, bl_self_ref, Wl_nb_ref, bl_nb_ref,
              Wl_e_ref, bl_e_ref, Wl_from_ref, Wl_to_ref,
              x_out_ref, e_out_ref):
    f32 = jnp.float32
    for s in range(IPP):
        node = node_ref[s]
        dem = dem_ref[s]
        M = m_ref[s][:NP, :NP]
        rowid = lax.broadcasted_iota(jnp.int32, (NP, NP), 0)
        colid = lax.broadcasted_iota(jnp.int32, (NP, NP), 1)
        a = jnp.where(rowid == colid, -1.0, M)

        xd = jax.nn.relu(jnp.dot(node, W1_ref[...],
                                 preferred_element_type=f32) + b1_ref[...])
        xc = jax.nn.relu(jnp.dot(node, W23_ref[...], preferred_element_type=f32)
                         + dem * w3_ref[...] + b23_ref[...])
        row0 = lax.broadcasted_iota(jnp.int32, (NP, 1), 0) == 0
        x = jnp.where(row0, xd, xc)
        x = jnp.dot(x, Wn_ref[...], preferred_element_type=f32)

        a_part = (a[:, :, None] * w5_ref[...][None]).reshape(NP * NP, H)
        e0 = jax.nn.relu(disf_ref[s] * w4_ref[...] + a_part + b45_ref[...])
        e = jnp.dot(e0, We_ref[...], preferred_element_type=f32)

        inv_k = 1.0 / K
        for l in range(L):
            mx = jnp.dot(M, x, preferred_element_type=f32) * inv_k
            agg = jnp.dot(mx, Wl_nb_ref[l], preferred_element_type=f32) + bl_nb_ref[l]
            x = x + jax.nn.relu(
                jnp.dot(x, Wl_self_ref[l], preferred_element_type=f32)
                + bl_self_ref[l] + agg)
            xf = jnp.dot(x, Wl_from_ref[l], preferred_element_type=f32) + bl_e_ref[l]
            xt = jnp.dot(x, Wl_to_ref[l], preferred_element_type=f32)
            et = jnp.dot(e, Wl_e_ref[l], preferred_element_type=f32)
            e3 = et.reshape(NP, NP, H) + xf[:, None, :] + xt[None, :, :]
            e = e + jax.nn.relu(e3).reshape(NP * NP, H)

        x_out_ref[s] = x[:N]
        e3 = e.reshape(NP, NP, H)
        for i in range(N):
            e_out_ref[pl.ds((s * N + i) * N, N), :] = e3[i, :N, :]


def kernel(node, demand, dis, W1, b1, W2, b2, W3, b3, W4, b4, W5, b5, Wn, We,
           Wl_self, bl_self, Wl_nb, bl_nb, Wl_e, bl_e, Wl_from, Wl_to):
    f32 = jnp.float32
    Hh = H // 2
    W23 = jnp.zeros((DIN, H), f32).at[:, :Hh].set(W2)
    w3 = jnp.zeros((1, H), f32).at[0, Hh:].set(W3[0])
    b23 = jnp.concatenate([b2, b3])[None]
    w4 = jnp.concatenate([W4[0], jnp.zeros((Hh,), f32)])[None]
    w5 = jnp.concatenate([jnp.zeros((Hh,), f32), W5[0]])[None]
    b45 = jnp.concatenate([b4, b5])[None]

    P = NP - N
    dis_p = jnp.pad(dis, ((0, 0), (0, P), (0, P)), constant_values=2.0)
    dis_flat = dis_p.reshape(B, NP * NP, 1)
    dis_sc = jnp.pad(dis_p, ((0, 0), (0, NI - NP), (0, NJ - NP)),
                     constant_values=2.0).reshape(NROWS, NJ)
    node_p = jnp.pad(node, ((0, 0), (0, P), (0, 0)))
    dem_p = jnp.pad(demand, ((0, 0), (0, P)))[..., None]

    (m_all,) = _sc_knn(dis_sc)
    m3 = m_all.reshape(B, NI, NJ)

    rep = lambda shape: pl.BlockSpec(shape, lambda b: (0,) * len(shape))

    grid_spec = pl.GridSpec(
        grid=(B // IPP,),
        in_specs=[
            pl.BlockSpec((IPP, NP, DIN), lambda b: (b, 0, 0)),
            pl.BlockSpec((IPP, NP, 1), lambda b: (b, 0, 0)),
            pl.BlockSpec((IPP, NI, NJ), lambda b: (b, 0, 0)),
            pl.BlockSpec((IPP, NP * NP, 1), lambda b: (b, 0, 0)),
            rep((DIN, H)), rep((1, H)), rep((DIN, H)), rep((1, H)),
            rep((1, H)), rep((1, H)), rep((1, H)), rep((1, H)),
            rep((H, H)), rep((H, H)),
            rep((L, H, H)), rep((L, H)), rep((L, H, H)), rep((L, H)),
            rep((L, H, H)), rep((L, H)), rep((L, H, H)), rep((L, H, H)),
        ],
        out_specs=[
            pl.BlockSpec((IPP, N, H), lambda b: (b, 0, 0)),
            pl.BlockSpec((IPP * N * N, H), lambda b: (b, 0)),
        ],
    )

    x_out, e_out = pl.pallas_call(
        _gcn_body,
        grid_spec=grid_spec,
        out_shape=[
            jax.ShapeDtypeStruct((B, N, H), f32),
            jax.ShapeDtypeStruct((B * N * N, H), f32),
        ],
        compiler_params=pltpu.CompilerParams(
            dimension_semantics=("arbitrary",),
        ),
    )(node_p, dem_p, m3, dis_flat, W1, b1[None], W23, w3, b23, w4, w5, b45,
      Wn, We, Wl_self, bl_self, Wl_nb, bl_nb, Wl_e, bl_e, Wl_from, Wl_to)
    return (x_out, e_out.reshape(B, N, N, H))


# speedup vs baseline: 1.4482x; 1.4292x over previous
import functools

import jax
import jax.numpy as jnp
from jax import lax
from jax.experimental import pallas as pl
from jax.experimental.pallas import tpu as pltpu
from jax.experimental.pallas import tpu_sc as plsc

B, N, DIN, H, L, K = 16, 100, 2, 128, 3, 10
NP = 104
NJ = 112
NI = 112
NCH = NJ // 16
NROWS = B * NI
NWORK = 32
RPW = NROWS // NWORK


def _splat_min(v):
    iota = lax.iota(jnp.int32, 16)
    for sh in (8, 4, 2, 1):
        idx = jnp.bitwise_xor(iota, sh)
        v = jnp.minimum(v, v.at[idx].get(mode="promise_in_bounds"))
    return v


def _sc_knn_body(dis_hbm, m_hbm, d_v, m_v):
    f32 = jnp.float32
    wid = lax.axis_index("s") * 2 + lax.axis_index("c")
    base = wid * RPW
    pltpu.sync_copy(dis_hbm.at[pl.ds(base, RPW)], d_v)

    iota = lax.iota(jnp.int32, 16)
    jids = [iota + (cc * 16) for cc in range(NCH)]

    UNROLL = 4

    def row_body(rr, carry):
        for u in range(UNROLL):
            r = rr * UNROLL + u
            vs = [d_v[r, pl.ds(cc * 16, 16)] for cc in range(NCH)]
            ms = [jnp.zeros((16,), f32) for _ in range(NCH)]
            for k in range(K + 1):
                mn = vs[0]
                for cc in range(1, NCH):
                    mn = jnp.minimum(mn, vs[cc])
                mval = _splat_min(mn)
                cands = [jnp.where(vs[cc] == mval, jids[cc], 9999)
                         for cc in range(NCH)]
                cm = cands[0]
                for cc in range(1, NCH):
                    cm = jnp.minimum(cm, cands[cc])
                jstar = _splat_min(cm)
                sels = [jids[cc] == jstar for cc in range(NCH)]
                vs = [jnp.where(sels[cc], 9.0, vs[cc]) for cc in range(NCH)]
                if k > 0:
                    ms = [jnp.where(sels[cc], 1.0, ms[cc])
                          for cc in range(NCH)]
            for cc in range(NCH):
                m_v[r, pl.ds(cc * 16, 16)] = ms[cc]
        return carry

    lax.fori_loop(0, RPW // UNROLL, row_body, 0)
    pltpu.sync_copy(m_v, m_hbm.at[pl.ds(base, RPW)])


def _sc_knn(dis_sc):
    f32 = jnp.float32
    mesh = plsc.VectorSubcoreMesh(core_axis_name="c", subcore_axis_name="s")
    fn = functools.partial(
        pl.kernel,
        mesh=mesh,
        out_type=[jax.ShapeDtypeStruct((NROWS, NJ), f32)],
        scratch_types=[pltpu.VMEM((RPW, NJ), f32),
                       pltpu.VMEM((RPW, NJ), f32)],
    )(_sc_knn_body)
    return fn(dis_sc)


IPP = 2


def _gcn_body(node_ref, dem_ref, m_ref, disf_ref, W1_ref, b1_ref,
              W23_ref, w3_ref, b23_ref, w4_ref, w5_ref, b45_ref, Wn_ref,
              We_ref, Wl_self_ref, bl_self_ref, Wl_nb_ref, bl_nb_ref,
              Wl_e_ref, bl_e_ref, Wl_from_ref, Wl_to_ref,
              x_out_ref, e_out_ref):
    f32 = jnp.float32
    for s in range(IPP):
        node = node_ref[s]
        dem = dem_ref[s]
        M = m_ref[s][:NP, :NP]
        rowid = lax.broadcasted_iota(jnp.int32, (NP, NP), 0)
        colid = lax.broadcasted_iota(jnp.int32, (NP, NP), 1)
        a = jnp.where(rowid == colid, -1.0, M)

        xd = jax.nn.relu(jnp.dot(node, W1_ref[...],
                                 preferred_element_type=f32) + b1_ref[...])
        xc = jax.nn.relu(jnp.dot(node, W23_ref[...], preferred_element_type=f32)
                         + dem * w3_ref[...] + b23_ref[...])
        row0 = lax.broadcasted_iota(jnp.int32, (NP, 1), 0) == 0
        x = jnp.where(row0, xd, xc)
        x = jnp.dot(x, Wn_ref[...], preferred_element_type=f32)

        a_part = (a[:, :, None] * w5_ref[...][None]).reshape(NP * NP, H)
        e0 = jax.nn.relu(disf_ref[s] * w4_ref[...] + a_part + b45_ref[...])
        e = jnp.dot(e0, We_ref[...], preferred_element_type=f32)

        inv_k = 1.0 / K
        for l in range(L):
            mx = jnp.dot(M, x, preferred_element_type=f32) * inv_k
            agg = jnp.dot(mx, Wl_nb_ref[l], preferred_element_type=f32) + bl_nb_ref[l]
            x = x + jax.nn.relu(
                jnp.dot(x, Wl_self_ref[l], preferred_element_type=f32)
                + bl_self_ref[l] + agg)
            xf = jnp.dot(x, Wl_from_ref[l], preferred_element_type=f32) + bl_e_ref[l]
            xt = jnp.dot(x, Wl_to_ref[l], preferred_element_type=f32)
            et = jnp.dot(e, Wl_e_ref[l], preferred_element_type=f32)
            e3 = et.reshape(NP, NP, H) + xf[:, None, :] + xt[None, :, :]
            e = e + jax.nn.relu(e3).reshape(NP * NP, H)

        x_out_ref[s] = x[:N]
        e_out_ref[s] = e.reshape(NP, NP, H)[:N, :N]


def kernel(node, demand, dis, W1, b1, W2, b2, W3, b3, W4, b4, W5, b5, Wn, We,
           Wl_self, bl_self, Wl_nb, bl_nb, Wl_e, bl_e, Wl_from, Wl_to):
    f32 = jnp.float32
    Hh = H // 2
    W23 = jnp.zeros((DIN, H), f32).at[:, :Hh].set(W2)
    w3 = jnp.zeros((1, H), f32).at[0, Hh:].set(W3[0])
    b23 = jnp.concatenate([b2, b3])[None]
    w4 = jnp.concatenate([W4[0], jnp.zeros((Hh,), f32)])[None]
    w5 = jnp.concatenate([jnp.zeros((Hh,), f32), W5[0]])[None]
    b45 = jnp.concatenate([b4, b5])[None]

    P = NP - N
    dis_p = jnp.pad(dis, ((0, 0), (0, P), (0, P)), constant_values=2.0)
    dis_flat = dis_p.reshape(B, NP * NP, 1)
    dis_sc = jnp.pad(dis_p, ((0, 0), (0, NI - NP), (0, NJ - NP)),
                     constant_values=2.0).reshape(NROWS, NJ)
    node_p = jnp.pad(node, ((0, 0), (0, P), (0, 0)))
    dem_p = jnp.pad(demand, ((0, 0), (0, P)))[..., None]

    (m_all,) = _sc_knn(dis_sc)
    m3 = m_all.reshape(B, NI, NJ)

    rep = lambda shape: pl.BlockSpec(shape, lambda b: (0,) * len(shape))

    grid_spec = pl.GridSpec(
        grid=(B // IPP,),
        in_specs=[
            pl.BlockSpec((IPP, NP, DIN), lambda b: (b, 0, 0)),
            pl.BlockSpec((IPP, NP, 1), lambda b: (b, 0, 0)),
            pl.BlockSpec((IPP, NI, NJ), lambda b: (b, 0, 0)),
            pl.BlockSpec((IPP, NP * NP, 1), lambda b: (b, 0, 0)),
            rep((DIN, H)), rep((1, H)), rep((DIN, H)), rep((1, H)),
            rep((1, H)), rep((1, H)), rep((1, H)), rep((1, H)),
            rep((H, H)), rep((H, H)),
            rep((L, H, H)), rep((L, H)), rep((L, H, H)), rep((L, H)),
            rep((L, H, H)), rep((L, H)), rep((L, H, H)), rep((L, H, H)),
        ],
        out_specs=[
            pl.BlockSpec((IPP, N, H), lambda b: (b, 0, 0)),
            pl.BlockSpec((IPP, N, N, H), lambda b: (b, 0, 0, 0)),
        ],
    )

    x_out, e_out = pl.pallas_call(
        _gcn_body,
        grid_spec=grid_spec,
        out_shape=[
            jax.ShapeDtypeStruct((B, N, H), f32),
            jax.ShapeDtypeStruct((B, N, N, H), f32),
        ],
        compiler_params=pltpu.CompilerParams(
            dimension_semantics=("arbitrary",),
        ),
    )(node_p, dem_p, m3, dis_flat, W1, b1[None], W23, w3, b23, w4, w5, b45,
      Wn, We, Wl_self, bl_self, Wl_nb, bl_nb, Wl_e, bl_e, Wl_from, Wl_to)
    return (x_out, e_out)


# speedup vs baseline: 1.4484x; 1.0001x over previous
import functools

import jax
import jax.numpy as jnp
from jax import lax
from jax.experimental import pallas as pl
from jax.experimental.pallas import tpu as pltpu
from jax.experimental.pallas import tpu_sc as plsc

B, N, DIN, H, L, K = 16, 100, 2, 128, 3, 10
NP = 104
NJ = 112
NI = 112
NCH = NJ // 16
NROWS = B * NI
NWORK = 32
RPW = NROWS // NWORK


def _splat_min(v):
    iota = lax.iota(jnp.int32, 16)
    for sh in (8, 4, 2, 1):
        idx = jnp.bitwise_xor(iota, sh)
        v = jnp.minimum(v, v.at[idx].get(mode="promise_in_bounds"))
    return v


def _sc_knn_body(dis_hbm, m_hbm, d_v, m_v):
    f32 = jnp.float32
    wid = lax.axis_index("s") * 2 + lax.axis_index("c")
    base = wid * RPW
    pltpu.sync_copy(dis_hbm.at[pl.ds(base, RPW)], d_v)

    iota = lax.iota(jnp.int32, 16)
    jids = [iota + (cc * 16) for cc in range(NCH)]

    UNROLL = 4

    def row_body(rr, carry):
        for u in range(UNROLL):
            r = rr * UNROLL + u
            vs = [d_v[r, pl.ds(cc * 16, 16)] for cc in range(NCH)]
            ms = [jnp.zeros((16,), f32) for _ in range(NCH)]
            for k in range(K + 1):
                mn = vs[0]
                for cc in range(1, NCH):
                    mn = jnp.minimum(mn, vs[cc])
                mval = _splat_min(mn)
                cands = [jnp.where(vs[cc] == mval, jids[cc], 9999)
                         for cc in range(NCH)]
                cm = cands[0]
                for cc in range(1, NCH):
                    cm = jnp.minimum(cm, cands[cc])
                jstar = _splat_min(cm)
                sels = [jids[cc] == jstar for cc in range(NCH)]
                vs = [jnp.where(sels[cc], 9.0, vs[cc]) for cc in range(NCH)]
                if k > 0:
                    ms = [jnp.where(sels[cc], 1.0, ms[cc])
                          for cc in range(NCH)]
            for cc in range(NCH):
                m_v[r, pl.ds(cc * 16, 16)] = ms[cc]
        return carry

    lax.fori_loop(0, RPW // UNROLL, row_body, 0)
    pltpu.sync_copy(m_v, m_hbm.at[pl.ds(base, RPW)])


def _sc_knn(dis_sc):
    f32 = jnp.float32
    mesh = plsc.VectorSubcoreMesh(core_axis_name="c", subcore_axis_name="s")
    fn = functools.partial(
        pl.kernel,
        mesh=mesh,
        out_type=[jax.ShapeDtypeStruct((NROWS, NJ), f32)],
        scratch_types=[pltpu.VMEM((RPW, NJ), f32),
                       pltpu.VMEM((RPW, NJ), f32)],
    )(_sc_knn_body)
    return fn(dis_sc)


IPP = 2


def _gcn_body(node_ref, dem_ref, m_ref, disf_ref, W1_ref, b1_ref,
              W23_ref, w3_ref, b23_ref, w4_ref, w5_ref, b45_ref, Wn_ref,
              We_ref, Wl_self_ref, bl_self_ref, Wl_nb_ref, bl_nb_ref,
              Wl_e_ref, bl_e_ref, Wl_from_ref, Wl_to_ref,
              x_out_ref, e_out_ref):
    f32 = jnp.float32
    for s in range(IPP):
        node = node_ref[s]
        dem = dem_ref[s]
        M = m_ref[s][:NP, :NP]
        rowid = lax.broadcasted_iota(jnp.int32, (NP, NP), 0)
        colid = lax.broadcasted_iota(jnp.int32, (NP, NP), 1)
        a = jnp.where(rowid == colid, -1.0, M)

        xd = jax.nn.relu(jnp.dot(node, W1_ref[...],
                                 preferred_element_type=f32) + b1_ref[...])
        xc = jax.nn.relu(jnp.dot(node, W23_ref[...], preferred_element_type=f32)
                         + dem * w3_ref[...] + b23_ref[...])
        row0 = lax.broadcasted_iota(jnp.int32, (NP, 1), 0) == 0
        x = jnp.where(row0, xd, xc)
        x = jnp.dot(x, Wn_ref[...], preferred_element_type=f32)

        a_part = (a[:, :, None] * w5_ref[...][None]).reshape(NP * NP, H)
        e0 = jax.nn.relu(disf_ref[s] * w4_ref[...] + a_part + b45_ref[...])
        e = jnp.dot(e0, We_ref[...], preferred_element_type=f32)

        inv_k = 1.0 / K
        for l in range(L):
            mx = jnp.dot(M, x, preferred_element_type=f32) * inv_k
            agg = jnp.dot(mx, Wl_nb_ref[l], preferred_element_type=f32) + bl_nb_ref[l]
            x = x + jax.nn.relu(
                jnp.dot(x, Wl_self_ref[l], preferred_element_type=f32)
                + bl_self_ref[l] + agg)
            xf = jnp.dot(x, Wl_from_ref[l], preferred_element_type=f32) + bl_e_ref[l]
            xt = jnp.dot(x, Wl_to_ref[l], preferred_element_type=f32)
            et = jnp.dot(e, Wl_e_ref[l], preferred_element_type=f32)
            e3 = et.reshape(NP, NP, H) + xf[:, None, :] + xt[None, :, :]
            e = e + jax.nn.relu(e3).reshape(NP * NP, H)

        x_out_ref[s] = x[:N]
        e_out_ref[s] = e.reshape(NP, NP, H)[:N, :N]


def kernel(node, demand, dis, W1, b1, W2, b2, W3, b3, W4, b4, W5, b5, Wn, We,
           Wl_self, bl_self, Wl_nb, bl_nb, Wl_e, bl_e, Wl_from, Wl_to):
    f32 = jnp.float32
    Hh = H // 2
    W23 = jnp.zeros((DIN, H), f32).at[:, :Hh].set(W2)
    w3 = jnp.zeros((1, H), f32).at[0, Hh:].set(W3[0])
    b23 = jnp.concatenate([b2, b3])[None]
    w4 = jnp.concatenate([W4[0], jnp.zeros((Hh,), f32)])[None]
    w5 = jnp.concatenate([jnp.zeros((Hh,), f32), W5[0]])[None]
    b45 = jnp.concatenate([b4, b5])[None]

    P = NP - N
    dis_p = jnp.pad(dis, ((0, 0), (0, P), (0, P)), constant_values=2.0)
    dis_flat = dis_p.reshape(B, NP * NP, 1)
    dis_sc = jnp.pad(dis_p, ((0, 0), (0, NI - NP), (0, NJ - NP)),
                     constant_values=2.0).reshape(NROWS, NJ)
    node_p = jnp.pad(node, ((0, 0), (0, P), (0, 0)))
    dem_p = jnp.pad(demand, ((0, 0), (0, P)))[..., None]

    (m_all,) = _sc_knn(dis_sc)
    m3 = m_all.reshape(B, NI, NJ)

    rep = lambda shape: pl.BlockSpec(shape, lambda b: (0,) * len(shape))

    grid_spec = pl.GridSpec(
        grid=(B // IPP,),
        in_specs=[
            pl.BlockSpec((IPP, NP, DIN), lambda b: (b, 0, 0)),
            pl.BlockSpec((IPP, NP, 1), lambda b: (b, 0, 0)),
            pl.BlockSpec((IPP, NI, NJ), lambda b: (b, 0, 0)),
            pl.BlockSpec((IPP, NP * NP, 1), lambda b: (b, 0, 0)),
            rep((DIN, H)), rep((1, H)), rep((DIN, H)), rep((1, H)),
            rep((1, H)), rep((1, H)), rep((1, H)), rep((1, H)),
            rep((H, H)), rep((H, H)),
            rep((L, H, H)), rep((L, H)), rep((L, H, H)), rep((L, H)),
            rep((L, H, H)), rep((L, H)), rep((L, H, H)), rep((L, H, H)),
        ],
        out_specs=[
            pl.BlockSpec((IPP, N, H), lambda b: (b, 0, 0)),
            pl.BlockSpec((IPP, N, N, H), lambda b: (b, 0, 0, 0)),
        ],
    )

    x_out, e_out = pl.pallas_call(
        _gcn_body,
        grid_spec=grid_spec,
        out_shape=[
            jax.ShapeDtypeStruct((B, N, H), f32),
            jax.ShapeDtypeStruct((B, N, N, H), f32),
        ],
        compiler_params=pltpu.CompilerParams(
            dimension_semantics=("parallel",),
        ),
    )(node_p, dem_p, m3, dis_flat, W1, b1[None], W23, w3, b23, w4, w5, b45,
      Wn, We, Wl_self, bl_self, Wl_nb, bl_nb, Wl_e, bl_e, Wl_from, Wl_to)
    return (x_out, e_out)
